# Initial kernel scaffold; baseline (speedup 1.0000x reference)
import functools

import jax
import jax.numpy as jnp
from jax import lax
from jax.experimental import pallas as pl
from jax.experimental.pallas import tpu as pltpu
from jax.experimental.pallas import tpu_sc as plsc

NUM_BINS = 61928
EMBED_DIM = 16
BATCH = 16384
SPAN = 8

_info = plsc.get_sparse_core_info()
NC, NS, L = _info.num_cores, _info.num_subcores, _info.num_lanes
NW = NC * NS

FEAT_PER_W = BATCH // NW
ROWS_PER_W = FEAT_PER_W * SPAN
CHUNK = 128
NCHUNK = ROWS_PER_W // CHUNK


def _make_kernel():
    mesh = plsc.VectorSubcoreMesh(core_axis_name="c", subcore_axis_name="s")

    @functools.partial(
        pl.kernel,
        mesh=mesh,
        out_type=jax.ShapeDtypeStruct((BATCH, EMBED_DIM), jnp.float32),
        scratch_types=[
            pltpu.VMEM((ROWS_PER_W,), jnp.int32),
            pltpu.VMEM((ROWS_PER_W, EMBED_DIM), jnp.float32),
            pltpu.VMEM((FEAT_PER_W, EMBED_DIM), jnp.float32),
            pltpu.SemaphoreType.DMA,
        ],
    )
    def k(idx_hbm, table_hbm, out_hbm, idx_v, rows_v, out_v, sem):
        wid = lax.axis_index("s") * NC + lax.axis_index("c")
        row_base = wid * ROWS_PER_W
        feat_base = wid * FEAT_PER_W

        pltpu.sync_copy(idx_hbm.at[pl.ds(row_base, ROWS_PER_W)], idx_v)

        def gather_body(j, carry):
            off = j * CHUNK
            pltpu.async_copy(
                table_hbm.at[idx_v.at[pl.ds(off, CHUNK)]],
                rows_v.at[pl.ds(off, CHUNK)],
                sem,
            ).wait()
            return carry

        lax.fori_loop(0, NCHUNK, gather_body, 0, unroll=False)

        inv = jnp.float32(1.0 / SPAN)

        def pool_body(f, carry):
            r = f * SPAN
            acc = rows_v[r, :]
            for s in range(1, SPAN):
                acc = acc + rows_v[r + s, :]
            out_v[f, :] = acc * inv
            return carry

        lax.fori_loop(0, FEAT_PER_W, pool_body, 0, unroll=False)

        pltpu.sync_copy(out_v, out_hbm.at[pl.ds(feat_base, FEAT_PER_W)])

    return k


_sc_kernel = _make_kernel()


def kernel(bin_idxs, table):
    idx_flat = bin_idxs.astype(jnp.int32).reshape(BATCH * SPAN)
    return _sc_kernel(idx_flat, table)


# speedup vs baseline: 5.0039x; 5.0039x over previous
import functools

import jax
import jax.numpy as jnp
from jax import lax
from jax.experimental import pallas as pl
from jax.experimental.pallas import tpu as pltpu
from jax.experimental.pallas import tpu_sc as plsc

NUM_BINS = 61928
EMBED_DIM = 16
BATCH = 16384
SPAN = 8

_info = plsc.get_sparse_core_info()
NC, NS, L = _info.num_cores, _info.num_subcores, _info.num_lanes
NW = NC * NS

FEAT_PER_W = BATCH // NW
ROWS_PER_W = FEAT_PER_W * SPAN
CHUNK = 128
NCHUNK = ROWS_PER_W // CHUNK


def _make_kernel():
    mesh = plsc.VectorSubcoreMesh(core_axis_name="c", subcore_axis_name="s")

    @functools.partial(
        pl.kernel,
        mesh=mesh,
        out_type=jax.ShapeDtypeStruct((BATCH, EMBED_DIM), jnp.float32),
        scratch_types=[
            pltpu.VMEM((ROWS_PER_W,), jnp.int32),
            pltpu.VMEM((ROWS_PER_W, EMBED_DIM), jnp.float32),
            pltpu.VMEM((FEAT_PER_W, EMBED_DIM), jnp.float32),
            pltpu.SemaphoreType.DMA,
        ],
        compiler_params=pltpu.CompilerParams(use_tc_tiling_on_sc=False),
    )
    def k(idx_hbm, table_hbm, out_hbm, idx_v, rows_v, out_v, sem):
        wid = lax.axis_index("s") * NC + lax.axis_index("c")
        row_base = wid * ROWS_PER_W
        feat_base = wid * FEAT_PER_W

        pltpu.sync_copy(idx_hbm.at[pl.ds(row_base, ROWS_PER_W)], idx_v)

        def gather_body(j, carry):
            off = j * CHUNK
            pltpu.async_copy(
                table_hbm.at[idx_v.at[pl.ds(off, CHUNK)]],
                rows_v.at[pl.ds(off, CHUNK)],
                sem,
            ).wait()
            return carry

        lax.fori_loop(0, NCHUNK, gather_body, 0, unroll=False)

        inv = jnp.float32(1.0 / SPAN)

        def pool_body(f, carry):
            r = f * SPAN
            acc = rows_v[r, :]
            for s in range(1, SPAN):
                acc = acc + rows_v[r + s, :]
            out_v[f, :] = acc * inv
            return carry

        lax.fori_loop(0, FEAT_PER_W, pool_body, 0, unroll=False)

        pltpu.sync_copy(out_v, out_hbm.at[pl.ds(feat_base, FEAT_PER_W)])

    return k


_sc_kernel = _make_kernel()


def kernel(bin_idxs, table):
    idx_flat = bin_idxs.astype(jnp.int32).reshape(BATCH * SPAN)
    return _sc_kernel(idx_flat, table)


# speedup vs baseline: 6.0181x; 1.2027x over previous
import functools

import jax
import jax.numpy as jnp
from jax import lax
from jax.experimental import pallas as pl
from jax.experimental.pallas import tpu as pltpu
from jax.experimental.pallas import tpu_sc as plsc

NUM_BINS = 61928
EMBED_DIM = 16
BATCH = 16384
SPAN = 8

_info = plsc.get_sparse_core_info()
NC, NS, L = _info.num_cores, _info.num_subcores, _info.num_lanes
NW = NC * NS

FEAT_PER_W = BATCH // NW
ROWS_PER_W = FEAT_PER_W * SPAN
CHUNK = 128
NCHUNK = ROWS_PER_W // CHUNK


def _make_kernel():
    mesh = plsc.VectorSubcoreMesh(core_axis_name="c", subcore_axis_name="s")

    @functools.partial(
        pl.kernel,
        mesh=mesh,
        out_type=jax.ShapeDtypeStruct((BATCH, EMBED_DIM), jnp.float32),
        scratch_types=[
            pltpu.VMEM((ROWS_PER_W,), jnp.int32),
            pltpu.VMEM((ROWS_PER_W, EMBED_DIM), jnp.float32),
            pltpu.VMEM((FEAT_PER_W, EMBED_DIM), jnp.float32),
            pltpu.SemaphoreType.DMA,
        ],
        compiler_params=pltpu.CompilerParams(use_tc_tiling_on_sc=False),
    )
    def k(idx_hbm, table_hbm, out_hbm, idx_v, rows_v, out_v, sem):
        wid = lax.axis_index("s") * NC + lax.axis_index("c")
        row_base = wid * ROWS_PER_W
        feat_base = wid * FEAT_PER_W

        pltpu.sync_copy(idx_hbm.at[pl.ds(row_base, ROWS_PER_W)], idx_v)

        def fire_body(j, carry):
            off = j * CHUNK
            pltpu.async_copy(
                table_hbm.at[idx_v.at[pl.ds(off, CHUNK)]],
                rows_v.at[pl.ds(off, CHUNK)],
                sem,
            )
            return carry

        lax.fori_loop(0, NCHUNK, fire_body, 0, unroll=False)

        def drain_body(j, carry):
            off = j * CHUNK
            pltpu.make_async_copy(
                table_hbm.at[idx_v.at[pl.ds(off, CHUNK)]],
                rows_v.at[pl.ds(off, CHUNK)],
                sem,
            ).wait()
            return carry

        lax.fori_loop(0, NCHUNK, drain_body, 0, unroll=False)

        inv = jnp.float32(1.0 / SPAN)

        def pool_body(f, carry):
            r = f * SPAN
            acc = rows_v[r, :]
            for s in range(1, SPAN):
                acc = acc + rows_v[r + s, :]
            out_v[f, :] = acc * inv
            return carry

        lax.fori_loop(0, FEAT_PER_W, pool_body, 0, unroll=False)

        pltpu.sync_copy(out_v, out_hbm.at[pl.ds(feat_base, FEAT_PER_W)])

    return k


_sc_kernel = _make_kernel()


def kernel(bin_idxs, table):
    idx_flat = bin_idxs.astype(jnp.int32).reshape(BATCH * SPAN)
    return _sc_kernel(idx_flat, table)
